# baseline (speedup 1.0000x reference)
import functools

import jax
import jax.numpy as jnp
from jax import lax
from jax.experimental import pallas as pl
from jax.experimental.pallas import tpu as pltpu
from jax.experimental.pallas import tpu_sc as plsc

NUM_WORKERS = 32
N = 800000
D = 64
NP = N // 2
CP = 400
NCHUNKS = NP // CP
PAIR_ITERS = CP // 16


def _sc_lookup(edge_type, table9):
    mesh = plsc.VectorSubcoreMesh(core_axis_name="c", subcore_axis_name="s")

    @functools.partial(
        pl.kernel,
        mesh=mesh,
        out_type=jax.ShapeDtypeStruct((NP, 2 * D), jnp.float32),
        scratch_types=[
            pltpu.VMEM((2 * CP,), jnp.int32),
            pltpu.VMEM((CP,), jnp.int32),
            pltpu.VMEM((CP, 2 * D), jnp.float32),
            pltpu.SemaphoreType.DMA,
        ],
    )
    def body(idx_hbm, t9_hbm, out_hbm, idx_v, pidx_v, rows_v, sem):
        wid = lax.axis_index("s") * 2 + lax.axis_index("c")
        nchunks_w = (NCHUNKS - wid + NUM_WORKERS - 1) // NUM_WORKERS
        lane = lax.iota(jnp.int32, 16)
        ev_lo = jnp.where(lane < 8, 2 * lane, 0)
        od_lo = jnp.where(lane < 8, 2 * lane + 1, 0)
        ev_hi = jnp.where(lane < 8, 0, 2 * lane - 16)
        od_hi = jnp.where(lane < 8, 0, 2 * lane - 15)

        dnums = lax.GatherDimensionNumbers(
            offset_dims=(), collapsed_slice_dims=(0,), start_index_map=(0,))

        def dg(v, i):
            return lax.gather(
                v, i[:, None], dnums, slice_sizes=(1,),
                mode=lax.GatherScatterMode.PROMISE_IN_BOUNDS)

        def step(k, carry):
            c = wid + k * NUM_WORKERS
            pltpu.sync_copy(idx_hbm.at[pl.ds(c * 2 * CP, 2 * CP)], idx_v)

            def pair_block(j, carry2):
                v0 = idx_v[pl.ds(32 * j, 16)]
                v1 = idx_v[pl.ds(32 * j + 16, 16)]
                p0 = 3 * dg(v0, ev_lo) + dg(v0, od_lo)
                p1 = 3 * dg(v1, ev_hi) + dg(v1, od_hi)
                pidx_v[pl.ds(16 * j, 16)] = jnp.where(lane < 8, p0, p1)
                return carry2

            lax.fori_loop(0, PAIR_ITERS, pair_block, 0)
            pltpu.async_copy(t9_hbm.at[pidx_v], rows_v, sem).wait()
            pltpu.sync_copy(rows_v, out_hbm.at[pl.ds(c * CP, CP)])
            return carry

        lax.fori_loop(0, nchunks_w, step, 0)

    return body(edge_type, table9)


def kernel(edge_type, table):
    table = table.astype(jnp.float32)
    table9 = jnp.concatenate(
        [jnp.repeat(table, 3, axis=0), jnp.tile(table, (3, 1))], axis=1)
    out2 = _sc_lookup(edge_type.astype(jnp.int32), table9)
    return out2.reshape(N, D)


# speedup vs baseline: 2.8573x; 2.8573x over previous
import functools

import jax
import jax.numpy as jnp
from jax import lax
from jax.experimental import pallas as pl
from jax.experimental.pallas import tpu as pltpu
from jax.experimental.pallas import tpu_sc as plsc

NUM_WORKERS = 32
N = 800000
D = 64
CR = 800
NCHUNKS = N // CR


def _sc_lookup(edge_type, table_flat):
    mesh = plsc.VectorSubcoreMesh(core_axis_name="c", subcore_axis_name="s")

    @functools.partial(
        pl.kernel,
        mesh=mesh,
        out_type=jax.ShapeDtypeStruct((N * D,), jnp.float32),
        scratch_types=[
            pltpu.VMEM((3 * D,), jnp.float32),
            pltpu.VMEM((CR,), jnp.int32),
            pltpu.VMEM((CR * D,), jnp.float32),
            pltpu.VMEM((CR * D,), jnp.float32),
            pltpu.SemaphoreType.DMA,
            pltpu.SemaphoreType.DMA,
        ],
    )
    def body(idx_hbm, tab_hbm, out_hbm, tab_v, idx_v, rows_a, rows_b,
             sem_a, sem_b):
        wid = lax.axis_index("s") * 2 + lax.axis_index("c")
        nchunks_w = (NCHUNKS - wid + NUM_WORKERS - 1) // NUM_WORKERS
        pltpu.sync_copy(tab_hbm, tab_v)

        def build(m, rows_v, sem):
            c = wid + m * NUM_WORKERS
            pltpu.sync_copy(idx_hbm.at[pl.ds(c * CR, CR)], idx_v)

            def row_block(b, carry):
                v = idx_v[pl.ds(16 * b, 16)] * D
                for j in range(16):
                    s = v[j]
                    r = (16 * b + j) * D
                    for k in range(D // 16):
                        rows_v[pl.ds(r + 16 * k, 16)] = (
                            tab_v[pl.ds(s + 16 * k, 16)])
                return carry

            lax.fori_loop(0, CR // 16, row_block, 0)
            pltpu.async_copy(
                rows_v, out_hbm.at[pl.ds(c * CR * D, CR * D)], sem)

        def drain(m, rows_v, sem):
            c = wid + m * NUM_WORKERS
            pltpu.make_async_copy(
                rows_v, out_hbm.at[pl.ds(c * CR * D, CR * D)], sem).wait()

        build(0, rows_a, sem_a)

        def step(k, carry):
            m1 = 2 * k + 1

            @pl.when(m1 < nchunks_w)
            def _():
                build(m1, rows_b, sem_b)

            drain(2 * k, rows_a, sem_a)
            m2 = 2 * k + 2

            @pl.when(m2 < nchunks_w)
            def _():
                build(m2, rows_a, sem_a)

            @pl.when(m1 < nchunks_w)
            def _():
                drain(m1, rows_b, sem_b)

            return carry

        lax.fori_loop(0, (nchunks_w + 1) // 2, step, 0)

    return body(edge_type, table_flat)


def kernel(edge_type, table):
    table_flat = table.astype(jnp.float32).reshape(3 * D)
    out = _sc_lookup(edge_type.astype(jnp.int32), table_flat)
    return out.reshape(N, D)


# speedup vs baseline: 3.5779x; 1.2522x over previous
import functools

import jax
import jax.numpy as jnp
from jax import lax
from jax.experimental import pallas as pl
from jax.experimental.pallas import tpu as pltpu
from jax.experimental.pallas import tpu_sc as plsc

NUM_WORKERS = 32
N = 800000
D = 64
RPW = N // NUM_WORKERS
CR = 200
NCH = RPW // CR


def _sc_lookup(edge_type, table_flat):
    mesh = plsc.VectorSubcoreMesh(core_axis_name="c", subcore_axis_name="s")

    @functools.partial(
        pl.kernel,
        mesh=mesh,
        out_type=jax.ShapeDtypeStruct((N, D), jnp.float32),
        scratch_types=[
            pltpu.VMEM((3 * D,), jnp.float32),
            pltpu.VMEM((CR,), jnp.int32),
            pltpu.VMEM((CR, D), jnp.float32),
            pltpu.VMEM((CR, D), jnp.float32),
            pltpu.SemaphoreType.DMA,
            pltpu.SemaphoreType.DMA,
        ],
    )
    def body(idx_hbm, tab_hbm, out_hbm, tab_v, idx_v, rows_a, rows_b,
             sem_a, sem_b):
        wid = lax.axis_index("s") * 2 + lax.axis_index("c")
        base = wid * RPW
        pltpu.sync_copy(tab_hbm, tab_v)

        def build(m, rows_v, sem):
            row0 = base + m * CR
            pltpu.sync_copy(idx_hbm.at[pl.ds(row0, CR)], idx_v)

            def row_block(b, carry):
                v = idx_v[pl.ds(16 * b, 16)] * D
                for j in range(16):
                    s = v[j]
                    r = 16 * b + j
                    for k in range(D // 16):
                        rows_v[r, pl.ds(16 * k, 16)] = (
                            tab_v[pl.ds(s + 16 * k, 16)])
                return carry

            lax.fori_loop(0, CR // 16, row_block, 0)
            pltpu.async_copy(rows_v, out_hbm.at[pl.ds(row0, CR)], sem)

        def drain(m, rows_v, sem):
            row0 = base + m * CR
            pltpu.make_async_copy(
                rows_v, out_hbm.at[pl.ds(row0, CR)], sem).wait()

        build(0, rows_a, sem_a)

        def step(k, carry):
            build(2 * k + 1, rows_b, sem_b)
            drain(2 * k, rows_a, sem_a)
            build(2 * k + 2, rows_a, sem_a)
            drain(2 * k + 1, rows_b, sem_b)
            return carry

        lax.fori_loop(0, (NCH - 1) // 2, step, 0)
        drain(NCH - 1, rows_a, sem_a)

    return body(edge_type, table_flat)


def kernel(edge_type, table):
    table_flat = table.astype(jnp.float32).reshape(3 * D)
    return _sc_lookup(edge_type.astype(jnp.int32), table_flat)
